# baseline (speedup 1.0000x reference)
import jax
import jax.numpy as jnp
from jax import lax
from jax.experimental import pallas as pl
from jax.experimental.pallas import tpu as pltpu
from jax.experimental.pallas import tpu_sc as plsc

DIMS = (8, 16, 32, 64, 128)
B = 16384
NC, NS = 2, 16
NW = NC * NS
BPW = B // NW
NCH = BPW // 128
V_BLK = 2048
N_BLK = 49
VPAD = N_BLK * V_BLK


def _tc_project_body(dw_ref, e0, e1, e2, e3, e4,
                     pw0, pw1, pw2, pw3, bstack_ref, out_ref):
    w = [dw_ref[i] for i in range(5)]
    dn = (((1,), (1,)), ((), ()))
    acc = w[4] * e4[...]
    acc += w[0] * lax.dot_general(e0[...], pw0[...], dn,
                                  preferred_element_type=jnp.float32)
    acc += w[1] * lax.dot_general(e1[...], pw1[...], dn,
                                  preferred_element_type=jnp.float32)
    acc += w[2] * lax.dot_general(e2[...], pw2[...], dn,
                                  preferred_element_type=jnp.float32)
    acc += w[3] * lax.dot_general(e3[...], pw3[...], dn,
                                  preferred_element_type=jnp.float32)
    bias = (w[0] * bstack_ref[0, :] + w[1] * bstack_ref[1, :]
            + w[2] * bstack_ref[2, :] + w[3] * bstack_ref[3, :])
    out_ref[...] = acc + bias[None, :]


def _tc_project(embs, pws, bstack, dw):
    return pl.pallas_call(
        _tc_project_body,
        grid=(N_BLK,),
        in_specs=[pl.BlockSpec(memory_space=pltpu.SMEM)]
        + [pl.BlockSpec((V_BLK, d), lambda i: (i, 0)) for d in DIMS]
        + [pl.BlockSpec((128, d), lambda i: (0, 0)) for d in DIMS[:-1]]
        + [pl.BlockSpec((4, 128), lambda i: (0, 0))],
        out_specs=pl.BlockSpec((V_BLK, 128), lambda i: (i, 0)),
        out_shape=jax.ShapeDtypeStruct((VPAD, 128), jnp.float32),
    )(dw, *embs, *pws, bstack)


def _sc_gather_body(x_hbm, wt, out_hbm, idx_v, rows_v, sem):
    wid = lax.axis_index("s") * NC + lax.axis_index("c")
    base = wid * BPW
    pltpu.sync_copy(x_hbm.at[pl.ds(wid * NCH, NCH)], idx_v)
    copies = [pltpu.async_copy(
        wt.at[idx_v.at[j]], rows_v.at[pl.ds(j * 128, 128)], sem)
        for j in range(NCH)]
    for c in copies:
        c.wait()
    pltpu.sync_copy(rows_v, out_hbm.at[pl.ds(base, BPW)])


def _sc_gather(x2d, wt):
    mesh = plsc.VectorSubcoreMesh(core_axis_name="c", subcore_axis_name="s")
    return pl.kernel(
        _sc_gather_body,
        out_type=jax.ShapeDtypeStruct((B, 128), jnp.float32),
        mesh=mesh,
        scratch_types=[
            pltpu.VMEM((NCH, 128), jnp.int32),
            pltpu.VMEM((BPW, 128), jnp.float32),
            pltpu.SemaphoreType.DMA,
        ],
        compiler_params=pltpu.CompilerParams(use_tc_tiling_on_sc=False),
    )(x2d, wt)


def kernel(x, emb_0, emb_1, emb_2, emb_3, emb_4,
           proj_w_0, proj_b_0, proj_w_1, proj_b_1,
           proj_w_2, proj_b_2, proj_w_3, proj_b_3,
           dim_logits):
    dim_weights = jax.nn.softmax(dim_logits, axis=-1)
    bstack = jnp.stack([proj_b_0, proj_b_1, proj_b_2, proj_b_3], axis=0)
    wide = _tc_project((emb_0, emb_1, emb_2, emb_3, emb_4),
                       (proj_w_0, proj_w_1, proj_w_2, proj_w_3),
                       bstack, dim_weights)
    x2d = x.astype(jnp.int32).reshape(B // 128, 128)
    out = _sc_gather(x2d, wide)
    return (out, dim_weights)
